# baseline (speedup 1.0000x reference)
import dataclasses

import jax
import jax.numpy as jnp
from jax import lax
from jax.experimental import pallas as pl
from jax.experimental.pallas import tpu as pltpu
from jax.experimental.pallas import tpu_sc as plsc

N = 10000
E = 320000
D = 128

NC = 2
NS = 16
K = 125
IROWS = E // K
IRPT = IROWS // (NC * NS)
EPT = E // (NC * NS)
SUBG = 8
NGRP = IRPT // SUBG

ROWS_A = 624
TAIL0 = ROWS_A * NS
TAIL = N - TAIL0

BLK = 1000
EPS = 1e-5


def _sc_mesh():
    return plsc.VectorSubcoreMesh(
        core_axis_name="c", subcore_axis_name="s", num_cores=NC, num_subcores=NS
    )


def _copy_node_rows(src_at, dst_at, s):
    pltpu.sync_copy(src_at(pl.ds(s * ROWS_A, ROWS_A)),
                    dst_at(pl.ds(s * ROWS_A, ROWS_A)))

    @pl.when(s == 0)
    def _():
        pltpu.sync_copy(src_at(pl.ds(TAIL0, TAIL)), dst_at(pl.ds(TAIL0, TAIL)))


NPAD = 10240
HROWS = NPAD // D


def _deg_kernel(dst1d, zeros80, lin80):
    def body(dst_hbm, zero_hbm, lin_hbm, out_hbm, acc, didx, hist, lin):
        c = lax.axis_index("c")
        s = lax.axis_index("s")
        wid = c * NS + s
        pltpu.sync_copy(dst_hbm.at[pl.ds(wid * EPT, EPT)], didx)
        pltpu.sync_copy(zero_hbm, hist)
        pltpu.sync_copy(lin_hbm, lin)

        @pl.when(s == 0)
        def _():
            pltpu.sync_copy(zero_hbm, acc)

        ones16 = jnp.ones((16,), jnp.float32)

        @pl.loop(0, EPT // 16)
        def _(i):
            idx16 = didx[pl.ds(i * 16, 16)]
            plsc.addupdate_scatter(
                hist, [idx16 >> 7, idx16 & 127], ones16)

        plsc.subcore_barrier()
        pltpu.sync_copy(hist, acc.at[lin], add=True)
        plsc.subcore_barrier()

        @pl.when(s == 0)
        def _():
            pltpu.sync_copy(acc, out_hbm.at[c])

    cp = pltpu.CompilerParams()
    if "needs_layout_passes" in pltpu.CompilerParams.__dataclass_fields__:
        cp = dataclasses.replace(cp, needs_layout_passes=False)
    f = pl.kernel(
        body,
        out_type=jax.ShapeDtypeStruct((NC, HROWS, D), jnp.float32),
        mesh=_sc_mesh(),
        compiler_params=cp,
        scratch_types=[
            pltpu.VMEM_SHARED((HROWS, D), jnp.float32),
            pltpu.VMEM((EPT,), jnp.int32),
            pltpu.VMEM((HROWS, D), jnp.float32),
            pltpu.VMEM((HROWS,), jnp.int32),
        ],
    )
    return f(dst1d, zeros80, lin80)


def _agg_kernel(g, src2d, dst2d):
    def body(g_hbm, src_hbm, dst_hbm, out_hbm, acc, sidx, didx, rows0, rows1,
             gsem0, gsem1, ssem0, ssem1, isem, initsem):
        c = lax.axis_index("c")
        s = lax.axis_index("s")
        wid = c * NS + s
        rowbuf = (rows0, rows1)
        gsem = (gsem0, gsem1)
        ssem = (ssem0, ssem1)

        def issue_idx(grp, p):
            row0 = wid * IRPT + grp * SUBG
            pltpu.async_copy(src_hbm.at[pl.ds(row0, SUBG)], sidx.at[p], isem)
            pltpu.async_copy(dst_hbm.at[pl.ds(row0, SUBG)], didx.at[p], isem)

        def wait_idx(p):
            pltpu.make_async_copy(
                src_hbm.at[pl.ds(0, SUBG)], sidx.at[p], isem).wait()
            pltpu.make_async_copy(
                dst_hbm.at[pl.ds(0, SUBG)], didx.at[p], isem).wait()

        def issue_gather(p, j, b):
            pltpu.async_copy(g_hbm.at[sidx.at[p].at[j]], rowbuf[b], gsem[b])

        def wait_gather(p, j, b):
            pltpu.make_async_copy(
                g_hbm.at[sidx.at[p].at[j]], rowbuf[b], gsem[b]).wait()

        def issue_scatter(p, j, b):
            pltpu.async_copy(
                rowbuf[b], acc.at[didx.at[p].at[j]], ssem[b], add=True)

        def wait_scatter(p, j, b):
            pltpu.make_async_copy(
                rowbuf[b], acc.at[didx.at[p].at[j]], ssem[b]).wait()

        init0 = pltpu.async_copy(g_hbm.at[pl.ds(s * ROWS_A, ROWS_A)],
                                 acc.at[pl.ds(s * ROWS_A, ROWS_A)], initsem)

        @pl.when(s == 0)
        def _():
            pltpu.async_copy(g_hbm.at[pl.ds(TAIL0, TAIL)],
                             acc.at[pl.ds(TAIL0, TAIL)], initsem)

        pltpu.sync_copy(src_hbm.at[pl.ds(wid * IRPT, SUBG)], sidx.at[0])
        pltpu.sync_copy(dst_hbm.at[pl.ds(wid * IRPT, SUBG)], didx.at[0])
        issue_gather(0, 0, 0)
        issue_gather(0, 1, 1)
        init0.wait()

        @pl.when(s == 0)
        def _():
            pltpu.make_async_copy(g_hbm.at[pl.ds(TAIL0, TAIL)],
                                  acc.at[pl.ds(TAIL0, TAIL)], initsem).wait()

        plsc.subcore_barrier()

        @pl.loop(0, NGRP // 2)
        def _(u):
            for p in range(2):
                for j in range(SUBG):
                    b = j % 2
                    bp = 1 - b
                    wait_gather(p, j, b)
                    issue_scatter(p, j, b)
                    if p == 0 and j == 0:
                        @pl.when(u > 0)
                        def _():
                            wait_scatter(1, SUBG - 1, bp)
                            issue_gather(0, 1, bp)
                        issue_idx(2 * u + 1, 1)
                    elif p == 1 and j == 0:
                        wait_scatter(0, SUBG - 1, bp)
                        issue_gather(1, 1, bp)

                        @pl.when(u < NGRP // 2 - 1)
                        def _():
                            issue_idx(2 * u + 2, 0)
                    elif p == 0 and j == SUBG - 1:
                        wait_scatter(0, j - 1, bp)
                        wait_idx(1)
                        issue_gather(1, 0, bp)
                    elif p == 1 and j == SUBG - 1:
                        @pl.when(u < NGRP // 2 - 1)
                        def _():
                            wait_scatter(1, j - 1, bp)
                            wait_idx(0)
                            issue_gather(0, 0, bp)
                    else:
                        wait_scatter(p, j - 1, bp)
                        issue_gather(p, j + 1, bp)

        wait_scatter(1, SUBG - 2, 0)
        wait_scatter(1, SUBG - 1, 1)

        plsc.subcore_barrier()
        _copy_node_rows(lambda d: acc.at[d], lambda d: out_hbm.at[c].at[d], s)

    f = pl.kernel(
        body,
        out_type=jax.ShapeDtypeStruct((NC, N, D), jnp.float32),
        mesh=_sc_mesh(),
        scratch_types=[
            pltpu.VMEM_SHARED((N, D), jnp.float32),
            pltpu.VMEM((2, SUBG, K), jnp.int32),
            pltpu.VMEM((2, SUBG, K), jnp.int32),
            pltpu.VMEM((K, D), jnp.float32),
            pltpu.VMEM((K, D), jnp.float32),
            pltpu.SemaphoreType.DMA,
            pltpu.SemaphoreType.DMA,
            pltpu.SemaphoreType.DMA,
            pltpu.SemaphoreType.DMA,
            pltpu.SemaphoreType.DMA,
            pltpu.SemaphoreType.DMA,
        ],
    )
    return f(g, src2d, dst2d)


_GRID = (N // BLK,)


def _row_spec(width=D):
    return pl.BlockSpec((BLK, width), lambda i: (i, 0))


def _full_spec(shape):
    return pl.BlockSpec(shape, lambda i: tuple(0 for _ in shape))


def _proj1(x, W1, d0, d1):
    def body(x_ref, w_ref, d0_ref, d1_ref, g_ref, dinv_ref):
        deg = d0_ref[...] + d1_ref[...] + 1.0
        dinv = lax.rsqrt(deg)
        dinv_ref[...] = dinv
        h = jnp.dot(x_ref[...], w_ref[...], preferred_element_type=jnp.float32)
        g_ref[...] = h * dinv

    return pl.pallas_call(
        body,
        grid=_GRID,
        in_specs=[_row_spec(), _full_spec((D, D)), _row_spec(1), _row_spec(1)],
        out_specs=[_row_spec(), _row_spec(1)],
        out_shape=[
            jax.ShapeDtypeStruct((N, D), jnp.float32),
            jax.ShapeDtypeStruct((N, 1), jnp.float32),
        ],
    )(x, W1, d0, d1)


def _p_spec():
    return pl.BlockSpec((NC, BLK, D), lambda *idx: (0, idx[-1], 0))


def _bn_layer(p, g, dinv, b, gamma, beta, W):
    def body(p_ref, g_ref, dinv_ref, b_ref, ga_ref, be_ref, w_ref, out_ref,
             z_ref, st_ref):
        ph = pl.program_id(0)
        i = pl.program_id(1)

        @pl.when(ph == 0)
        def _():
            z = dinv_ref[...] * (p_ref[0] + p_ref[1] - g_ref[...]) + b_ref[...]
            z_ref[pl.ds(i * BLK, BLK), :] = z

            @pl.when(i == 0)
            def _():
                st_ref[...] = jnp.zeros_like(st_ref)

            st_ref[0:1, :] += jnp.sum(z, axis=0, keepdims=True)
            st_ref[1:2, :] += jnp.sum(z * z, axis=0, keepdims=True)

        @pl.when(ph == 1)
        def _():
            mu = st_ref[0:1, :] * (1.0 / N)
            var = st_ref[1:2, :] * (1.0 / N) - mu * mu
            z = z_ref[pl.ds(i * BLK, BLK), :]
            t = (z - mu) * lax.rsqrt(var + EPS) * ga_ref[...] + be_ref[...]
            t = jnp.maximum(t, 0.0)
            h = jnp.dot(t, w_ref[...], preferred_element_type=jnp.float32)
            out_ref[...] = h * dinv_ref[...]

    return pl.pallas_call(
        body,
        grid=(2,) + _GRID,
        in_specs=[_p_spec(),
                  pl.BlockSpec((BLK, D), lambda ph, i: (i, 0)),
                  pl.BlockSpec((BLK, 1), lambda ph, i: (i, 0)),
                  pl.BlockSpec((1, D), lambda ph, i: (0, 0)),
                  pl.BlockSpec((1, D), lambda ph, i: (0, 0)),
                  pl.BlockSpec((1, D), lambda ph, i: (0, 0)),
                  pl.BlockSpec((D, D), lambda ph, i: (0, 0))],
        out_specs=pl.BlockSpec((BLK, D), lambda ph, i: (i, 0)),
        out_shape=jax.ShapeDtypeStruct((N, D), jnp.float32),
        scratch_shapes=[
            pltpu.VMEM((N, D), jnp.float32),
            pltpu.VMEM((8, D), jnp.float32),
        ],
    )(p, g, dinv, b, gamma, beta, W)


def _final(p, g, dinv, b):
    def body(p_ref, g_ref, dinv_ref, b_ref, o_ref):
        z = dinv_ref[...] * (p_ref[0] + p_ref[1] - g_ref[...]) + b_ref[...]
        o_ref[...] = jax.nn.sigmoid(z)

    return pl.pallas_call(
        body,
        grid=_GRID,
        in_specs=[_p_spec(), _row_spec(), _row_spec(1), _full_spec((1, D))],
        out_specs=_row_spec(),
        out_shape=jax.ShapeDtypeStruct((N, D), jnp.float32),
    )(p, g, dinv, b)


def kernel(x, edge_index, W1, b1, gamma1, beta1, W2, b2, gamma2, beta2, W3, b3):
    src2d = edge_index[0].reshape(IROWS, K)
    dst2d = edge_index[1].reshape(IROWS, K)
    zeros80 = jnp.zeros((HROWS, D), jnp.float32)
    lin80 = jnp.arange(HROWS, dtype=jnp.int32)
    b1r, b2r, b3r = (v.reshape(1, D) for v in (b1, b2, b3))
    ga1, be1 = gamma1.reshape(1, D), beta1.reshape(1, D)
    ga2, be2 = gamma2.reshape(1, D), beta2.reshape(1, D)

    dp = _deg_kernel(edge_index[1], zeros80, lin80)
    d0 = dp[0].reshape(NPAD, 1)[:N]
    d1 = dp[1].reshape(NPAD, 1)[:N]
    g1, dinv = _proj1(x, W1, d0, d1)

    p = _agg_kernel(g1, src2d, dst2d)
    g2 = _bn_layer(p, g1, dinv, b1r, ga1, be1, W2)

    p = _agg_kernel(g2, src2d, dst2d)
    g3 = _bn_layer(p, g2, dinv, b2r, ga2, be2, W3)

    p = _agg_kernel(g3, src2d, dst2d)
    return _final(p, g3, dinv, b3r)


# speedup vs baseline: 1.0273x; 1.0273x over previous
import dataclasses

import jax
import jax.numpy as jnp
from jax import lax
from jax.experimental import pallas as pl
from jax.experimental.pallas import tpu as pltpu
from jax.experimental.pallas import tpu_sc as plsc

N = 10000
E = 320000
D = 128

NC = 2
NS = 16
K = 125
IROWS = E // K
IRPT = IROWS // (NC * NS)
EPT = E // (NC * NS)
SUBG = 8
NGRP = IRPT // SUBG

ROWS_A = 624
TAIL0 = ROWS_A * NS
TAIL = N - TAIL0

BLK = 2000
EPS = 1e-5


def _sc_mesh():
    return plsc.VectorSubcoreMesh(
        core_axis_name="c", subcore_axis_name="s", num_cores=NC, num_subcores=NS
    )


def _copy_node_rows(src_at, dst_at, s):
    pltpu.sync_copy(src_at(pl.ds(s * ROWS_A, ROWS_A)),
                    dst_at(pl.ds(s * ROWS_A, ROWS_A)))

    @pl.when(s == 0)
    def _():
        pltpu.sync_copy(src_at(pl.ds(TAIL0, TAIL)), dst_at(pl.ds(TAIL0, TAIL)))


NPAD = 10240
HROWS = NPAD // D


def _deg_kernel(dst1d, zeros80, lin80):
    def body(dst_hbm, zero_hbm, lin_hbm, out_hbm, acc, didx, hist, lin):
        c = lax.axis_index("c")
        s = lax.axis_index("s")
        wid = c * NS + s
        pltpu.sync_copy(dst_hbm.at[pl.ds(wid * EPT, EPT)], didx)
        pltpu.sync_copy(zero_hbm, hist)
        pltpu.sync_copy(lin_hbm, lin)

        @pl.when(s == 0)
        def _():
            pltpu.sync_copy(zero_hbm, acc)

        ones16 = jnp.ones((16,), jnp.float32)

        @pl.loop(0, EPT // 16)
        def _(i):
            idx16 = didx[pl.ds(i * 16, 16)]
            plsc.addupdate_scatter(
                hist, [idx16 >> 7, idx16 & 127], ones16)

        plsc.subcore_barrier()
        pltpu.sync_copy(hist, acc.at[lin], add=True)
        plsc.subcore_barrier()

        @pl.when(s == 0)
        def _():
            pltpu.sync_copy(acc, out_hbm.at[c])

    cp = pltpu.CompilerParams()
    if "needs_layout_passes" in pltpu.CompilerParams.__dataclass_fields__:
        cp = dataclasses.replace(cp, needs_layout_passes=False)
    f = pl.kernel(
        body,
        out_type=jax.ShapeDtypeStruct((NC, HROWS, D), jnp.float32),
        mesh=_sc_mesh(),
        compiler_params=cp,
        scratch_types=[
            pltpu.VMEM_SHARED((HROWS, D), jnp.float32),
            pltpu.VMEM((EPT,), jnp.int32),
            pltpu.VMEM((HROWS, D), jnp.float32),
            pltpu.VMEM((HROWS,), jnp.int32),
        ],
    )
    return f(dst1d, zeros80, lin80)


def _agg_kernel(g, src2d, dst2d):
    def body(g_hbm, src_hbm, dst_hbm, out_hbm, acc, sidx, didx, rows0, rows1,
             gsem0, gsem1, ssem0, ssem1, isem, initsem):
        c = lax.axis_index("c")
        s = lax.axis_index("s")
        wid = c * NS + s
        rowbuf = (rows0, rows1)
        gsem = (gsem0, gsem1)
        ssem = (ssem0, ssem1)

        def issue_idx(grp, p):
            row0 = wid * IRPT + grp * SUBG
            pltpu.async_copy(src_hbm.at[pl.ds(row0, SUBG)], sidx.at[p], isem)
            pltpu.async_copy(dst_hbm.at[pl.ds(row0, SUBG)], didx.at[p], isem)

        def wait_idx(p):
            pltpu.make_async_copy(
                src_hbm.at[pl.ds(0, SUBG)], sidx.at[p], isem).wait()
            pltpu.make_async_copy(
                dst_hbm.at[pl.ds(0, SUBG)], didx.at[p], isem).wait()

        def issue_gather(p, j, b):
            pltpu.async_copy(g_hbm.at[sidx.at[p].at[j]], rowbuf[b], gsem[b])

        def wait_gather(p, j, b):
            pltpu.make_async_copy(
                g_hbm.at[sidx.at[p].at[j]], rowbuf[b], gsem[b]).wait()

        def issue_scatter(p, j, b):
            pltpu.async_copy(
                rowbuf[b], acc.at[didx.at[p].at[j]], ssem[b], add=True)

        def wait_scatter(p, j, b):
            pltpu.make_async_copy(
                rowbuf[b], acc.at[didx.at[p].at[j]], ssem[b]).wait()

        init0 = pltpu.async_copy(g_hbm.at[pl.ds(s * ROWS_A, ROWS_A)],
                                 acc.at[pl.ds(s * ROWS_A, ROWS_A)], initsem)

        @pl.when(s == 0)
        def _():
            pltpu.async_copy(g_hbm.at[pl.ds(TAIL0, TAIL)],
                             acc.at[pl.ds(TAIL0, TAIL)], initsem)

        pltpu.sync_copy(src_hbm.at[pl.ds(wid * IRPT, SUBG)], sidx.at[0])
        pltpu.sync_copy(dst_hbm.at[pl.ds(wid * IRPT, SUBG)], didx.at[0])
        issue_gather(0, 0, 0)
        issue_gather(0, 1, 1)
        init0.wait()

        @pl.when(s == 0)
        def _():
            pltpu.make_async_copy(g_hbm.at[pl.ds(TAIL0, TAIL)],
                                  acc.at[pl.ds(TAIL0, TAIL)], initsem).wait()

        plsc.subcore_barrier()

        @pl.loop(0, NGRP // 2)
        def _(u):
            for p in range(2):
                for j in range(SUBG):
                    b = j % 2
                    bp = 1 - b
                    wait_gather(p, j, b)
                    issue_scatter(p, j, b)
                    if p == 0 and j == 0:
                        @pl.when(u > 0)
                        def _():
                            wait_scatter(1, SUBG - 1, bp)
                            issue_gather(0, 1, bp)
                        issue_idx(2 * u + 1, 1)
                    elif p == 1 and j == 0:
                        wait_scatter(0, SUBG - 1, bp)
                        issue_gather(1, 1, bp)

                        @pl.when(u < NGRP // 2 - 1)
                        def _():
                            issue_idx(2 * u + 2, 0)
                    elif p == 0 and j == SUBG - 1:
                        wait_scatter(0, j - 1, bp)
                        wait_idx(1)
                        issue_gather(1, 0, bp)
                    elif p == 1 and j == SUBG - 1:
                        @pl.when(u < NGRP // 2 - 1)
                        def _():
                            wait_scatter(1, j - 1, bp)
                            wait_idx(0)
                            issue_gather(0, 0, bp)
                    else:
                        wait_scatter(p, j - 1, bp)
                        issue_gather(p, j + 1, bp)

        wait_scatter(1, SUBG - 2, 0)
        wait_scatter(1, SUBG - 1, 1)

        plsc.subcore_barrier()
        _copy_node_rows(lambda d: acc.at[d], lambda d: out_hbm.at[c].at[d], s)

    f = pl.kernel(
        body,
        out_type=jax.ShapeDtypeStruct((NC, N, D), jnp.float32),
        mesh=_sc_mesh(),
        scratch_types=[
            pltpu.VMEM_SHARED((N, D), jnp.float32),
            pltpu.VMEM((2, SUBG, K), jnp.int32),
            pltpu.VMEM((2, SUBG, K), jnp.int32),
            pltpu.VMEM((K, D), jnp.float32),
            pltpu.VMEM((K, D), jnp.float32),
            pltpu.SemaphoreType.DMA,
            pltpu.SemaphoreType.DMA,
            pltpu.SemaphoreType.DMA,
            pltpu.SemaphoreType.DMA,
            pltpu.SemaphoreType.DMA,
            pltpu.SemaphoreType.DMA,
        ],
    )
    return f(g, src2d, dst2d)


_GRID = (N // BLK,)


def _row_spec(width=D):
    return pl.BlockSpec((BLK, width), lambda i: (i, 0))


def _full_spec(shape):
    return pl.BlockSpec(shape, lambda i: tuple(0 for _ in shape))


def _proj1(x, W1, d0, d1):
    def body(x_ref, w_ref, d0_ref, d1_ref, g_ref, dinv_ref):
        deg = d0_ref[...] + d1_ref[...] + 1.0
        dinv = lax.rsqrt(deg)
        dinv_ref[...] = dinv
        h = jnp.dot(x_ref[...], w_ref[...], preferred_element_type=jnp.float32)
        g_ref[...] = h * dinv

    return pl.pallas_call(
        body,
        grid=_GRID,
        in_specs=[_row_spec(), _full_spec((D, D)), _row_spec(1), _row_spec(1)],
        out_specs=[_row_spec(), _row_spec(1)],
        out_shape=[
            jax.ShapeDtypeStruct((N, D), jnp.float32),
            jax.ShapeDtypeStruct((N, 1), jnp.float32),
        ],
    )(x, W1, d0, d1)


def _p_spec():
    return pl.BlockSpec((NC, BLK, D), lambda *idx: (0, idx[-1], 0))


def _bn_layer(p, g, dinv, b, gamma, beta, W):
    def body(p_ref, g_ref, dinv_ref, b_ref, ga_ref, be_ref, w_ref, out_ref,
             z_ref, st_ref):
        ph = pl.program_id(0)
        i = pl.program_id(1)

        @pl.when(ph == 0)
        def _():
            z = dinv_ref[...] * (p_ref[0] + p_ref[1] - g_ref[...]) + b_ref[...]
            z_ref[pl.ds(i * BLK, BLK), :] = z

            @pl.when(i == 0)
            def _():
                st_ref[...] = jnp.zeros_like(st_ref)

            st_ref[0:1, :] += jnp.sum(z, axis=0, keepdims=True)
            st_ref[1:2, :] += jnp.sum(z * z, axis=0, keepdims=True)

        @pl.when(ph == 1)
        def _():
            mu = st_ref[0:1, :] * (1.0 / N)
            var = st_ref[1:2, :] * (1.0 / N) - mu * mu
            z = z_ref[pl.ds(i * BLK, BLK), :]
            t = (z - mu) * lax.rsqrt(var + EPS) * ga_ref[...] + be_ref[...]
            t = jnp.maximum(t, 0.0)
            h = jnp.dot(t, w_ref[...], preferred_element_type=jnp.float32)
            out_ref[...] = h * dinv_ref[...]

    return pl.pallas_call(
        body,
        grid=(2,) + _GRID,
        in_specs=[_p_spec(),
                  pl.BlockSpec((BLK, D), lambda ph, i: (i, 0)),
                  pl.BlockSpec((BLK, 1), lambda ph, i: (i, 0)),
                  pl.BlockSpec((1, D), lambda ph, i: (0, 0)),
                  pl.BlockSpec((1, D), lambda ph, i: (0, 0)),
                  pl.BlockSpec((1, D), lambda ph, i: (0, 0)),
                  pl.BlockSpec((D, D), lambda ph, i: (0, 0))],
        out_specs=pl.BlockSpec((BLK, D), lambda ph, i: (i, 0)),
        out_shape=jax.ShapeDtypeStruct((N, D), jnp.float32),
        scratch_shapes=[
            pltpu.VMEM((N, D), jnp.float32),
            pltpu.VMEM((8, D), jnp.float32),
        ],
    )(p, g, dinv, b, gamma, beta, W)


def _final(p, g, dinv, b):
    def body(p_ref, g_ref, dinv_ref, b_ref, o_ref):
        z = dinv_ref[...] * (p_ref[0] + p_ref[1] - g_ref[...]) + b_ref[...]
        o_ref[...] = jax.nn.sigmoid(z)

    return pl.pallas_call(
        body,
        grid=_GRID,
        in_specs=[_p_spec(), _row_spec(), _row_spec(1), _full_spec((1, D))],
        out_specs=_row_spec(),
        out_shape=jax.ShapeDtypeStruct((N, D), jnp.float32),
    )(p, g, dinv, b)


def kernel(x, edge_index, W1, b1, gamma1, beta1, W2, b2, gamma2, beta2, W3, b3):
    src2d = edge_index[0].reshape(IROWS, K)
    dst2d = edge_index[1].reshape(IROWS, K)
    zeros80 = jnp.zeros((HROWS, D), jnp.float32)
    lin80 = jnp.arange(HROWS, dtype=jnp.int32)
    b1r, b2r, b3r = (v.reshape(1, D) for v in (b1, b2, b3))
    ga1, be1 = gamma1.reshape(1, D), beta1.reshape(1, D)
    ga2, be2 = gamma2.reshape(1, D), beta2.reshape(1, D)

    dp = _deg_kernel(edge_index[1], zeros80, lin80)
    d0 = dp[0].reshape(NPAD, 1)[:N]
    d1 = dp[1].reshape(NPAD, 1)[:N]
    g1, dinv = _proj1(x, W1, d0, d1)

    p = _agg_kernel(g1, src2d, dst2d)
    g2 = _bn_layer(p, g1, dinv, b1r, ga1, be1, W2)

    p = _agg_kernel(g2, src2d, dst2d)
    g3 = _bn_layer(p, g2, dinv, b2r, ga2, be2, W3)

    p = _agg_kernel(g3, src2d, dst2d)
    return _final(p, g3, dinv, b3r)


# speedup vs baseline: 1.0356x; 1.0080x over previous
import dataclasses

import jax
import jax.numpy as jnp
from jax import lax
from jax.experimental import pallas as pl
from jax.experimental.pallas import tpu as pltpu
from jax.experimental.pallas import tpu_sc as plsc

N = 10000
E = 320000
D = 128

NC = 2
NS = 16
K = 125
IROWS = E // K
IRPT = IROWS // (NC * NS)
EPT = E // (NC * NS)
SUBG = 8
NGRP = IRPT // SUBG

ROWS_A = 624
TAIL0 = ROWS_A * NS
TAIL = N - TAIL0

BLK = 5000
EPS = 1e-5


def _sc_mesh():
    return plsc.VectorSubcoreMesh(
        core_axis_name="c", subcore_axis_name="s", num_cores=NC, num_subcores=NS
    )


def _copy_node_rows(src_at, dst_at, s):
    pltpu.sync_copy(src_at(pl.ds(s * ROWS_A, ROWS_A)),
                    dst_at(pl.ds(s * ROWS_A, ROWS_A)))

    @pl.when(s == 0)
    def _():
        pltpu.sync_copy(src_at(pl.ds(TAIL0, TAIL)), dst_at(pl.ds(TAIL0, TAIL)))


NPAD = 10240
HROWS = NPAD // D


def _deg_kernel(dst1d, zeros80, lin80):
    def body(dst_hbm, zero_hbm, lin_hbm, out_hbm, acc, didx, hist, lin):
        c = lax.axis_index("c")
        s = lax.axis_index("s")
        wid = c * NS + s
        pltpu.sync_copy(dst_hbm.at[pl.ds(wid * EPT, EPT)], didx)
        pltpu.sync_copy(zero_hbm, hist)
        pltpu.sync_copy(lin_hbm, lin)

        @pl.when(s == 0)
        def _():
            pltpu.sync_copy(zero_hbm, acc)

        ones16 = jnp.ones((16,), jnp.float32)

        @pl.loop(0, EPT // 16)
        def _(i):
            idx16 = didx[pl.ds(i * 16, 16)]
            plsc.addupdate_scatter(
                hist, [idx16 >> 7, idx16 & 127], ones16)

        plsc.subcore_barrier()
        pltpu.sync_copy(hist, acc.at[lin], add=True)
        plsc.subcore_barrier()

        @pl.when(s == 0)
        def _():
            pltpu.sync_copy(acc, out_hbm.at[c])

    cp = pltpu.CompilerParams()
    if "needs_layout_passes" in pltpu.CompilerParams.__dataclass_fields__:
        cp = dataclasses.replace(cp, needs_layout_passes=False)
    f = pl.kernel(
        body,
        out_type=jax.ShapeDtypeStruct((NC, HROWS, D), jnp.float32),
        mesh=_sc_mesh(),
        compiler_params=cp,
        scratch_types=[
            pltpu.VMEM_SHARED((HROWS, D), jnp.float32),
            pltpu.VMEM((EPT,), jnp.int32),
            pltpu.VMEM((HROWS, D), jnp.float32),
            pltpu.VMEM((HROWS,), jnp.int32),
        ],
    )
    return f(dst1d, zeros80, lin80)


def _agg_kernel(g, src2d, dst2d):
    def body(g_hbm, src_hbm, dst_hbm, out_hbm, acc, sidx, didx, rows0, rows1,
             gsem0, gsem1, ssem0, ssem1, isem, initsem):
        c = lax.axis_index("c")
        s = lax.axis_index("s")
        wid = c * NS + s
        rowbuf = (rows0, rows1)
        gsem = (gsem0, gsem1)
        ssem = (ssem0, ssem1)

        def issue_idx(grp, p):
            row0 = wid * IRPT + grp * SUBG
            pltpu.async_copy(src_hbm.at[pl.ds(row0, SUBG)], sidx.at[p], isem)
            pltpu.async_copy(dst_hbm.at[pl.ds(row0, SUBG)], didx.at[p], isem)

        def wait_idx(p):
            pltpu.make_async_copy(
                src_hbm.at[pl.ds(0, SUBG)], sidx.at[p], isem).wait()
            pltpu.make_async_copy(
                dst_hbm.at[pl.ds(0, SUBG)], didx.at[p], isem).wait()

        def issue_gather(p, j, b):
            pltpu.async_copy(g_hbm.at[sidx.at[p].at[j]], rowbuf[b], gsem[b])

        def wait_gather(p, j, b):
            pltpu.make_async_copy(
                g_hbm.at[sidx.at[p].at[j]], rowbuf[b], gsem[b]).wait()

        def issue_scatter(p, j, b):
            pltpu.async_copy(
                rowbuf[b], acc.at[didx.at[p].at[j]], ssem[b], add=True)

        def wait_scatter(p, j, b):
            pltpu.make_async_copy(
                rowbuf[b], acc.at[didx.at[p].at[j]], ssem[b]).wait()

        init0 = pltpu.async_copy(g_hbm.at[pl.ds(s * ROWS_A, ROWS_A)],
                                 acc.at[pl.ds(s * ROWS_A, ROWS_A)], initsem)

        @pl.when(s == 0)
        def _():
            pltpu.async_copy(g_hbm.at[pl.ds(TAIL0, TAIL)],
                             acc.at[pl.ds(TAIL0, TAIL)], initsem)

        pltpu.sync_copy(src_hbm.at[pl.ds(wid * IRPT, SUBG)], sidx.at[0])
        pltpu.sync_copy(dst_hbm.at[pl.ds(wid * IRPT, SUBG)], didx.at[0])
        issue_gather(0, 0, 0)
        issue_gather(0, 1, 1)
        init0.wait()

        @pl.when(s == 0)
        def _():
            pltpu.make_async_copy(g_hbm.at[pl.ds(TAIL0, TAIL)],
                                  acc.at[pl.ds(TAIL0, TAIL)], initsem).wait()

        plsc.subcore_barrier()

        @pl.loop(0, NGRP // 2)
        def _(u):
            for p in range(2):
                for j in range(SUBG):
                    b = j % 2
                    bp = 1 - b
                    wait_gather(p, j, b)
                    issue_scatter(p, j, b)
                    if p == 0 and j == 0:
                        @pl.when(u > 0)
                        def _():
                            wait_scatter(1, SUBG - 1, bp)
                            issue_gather(0, 1, bp)
                        issue_idx(2 * u + 1, 1)
                    elif p == 1 and j == 0:
                        wait_scatter(0, SUBG - 1, bp)
                        issue_gather(1, 1, bp)

                        @pl.when(u < NGRP // 2 - 1)
                        def _():
                            issue_idx(2 * u + 2, 0)
                    elif p == 0 and j == SUBG - 1:
                        wait_scatter(0, j - 1, bp)
                        wait_idx(1)
                        issue_gather(1, 0, bp)
                    elif p == 1 and j == SUBG - 1:
                        @pl.when(u < NGRP // 2 - 1)
                        def _():
                            wait_scatter(1, j - 1, bp)
                            wait_idx(0)
                            issue_gather(0, 0, bp)
                    else:
                        wait_scatter(p, j - 1, bp)
                        issue_gather(p, j + 1, bp)

        wait_scatter(1, SUBG - 2, 0)
        wait_scatter(1, SUBG - 1, 1)

        plsc.subcore_barrier()
        _copy_node_rows(lambda d: acc.at[d], lambda d: out_hbm.at[c].at[d], s)

    f = pl.kernel(
        body,
        out_type=jax.ShapeDtypeStruct((NC, N, D), jnp.float32),
        mesh=_sc_mesh(),
        scratch_types=[
            pltpu.VMEM_SHARED((N, D), jnp.float32),
            pltpu.VMEM((2, SUBG, K), jnp.int32),
            pltpu.VMEM((2, SUBG, K), jnp.int32),
            pltpu.VMEM((K, D), jnp.float32),
            pltpu.VMEM((K, D), jnp.float32),
            pltpu.SemaphoreType.DMA,
            pltpu.SemaphoreType.DMA,
            pltpu.SemaphoreType.DMA,
            pltpu.SemaphoreType.DMA,
            pltpu.SemaphoreType.DMA,
            pltpu.SemaphoreType.DMA,
        ],
    )
    return f(g, src2d, dst2d)


_GRID = (N // BLK,)


def _row_spec(width=D):
    return pl.BlockSpec((BLK, width), lambda i: (i, 0))


def _full_spec(shape):
    return pl.BlockSpec(shape, lambda i: tuple(0 for _ in shape))


def _proj1(x, W1, d0, d1):
    def body(x_ref, w_ref, d0_ref, d1_ref, g_ref, dinv_ref):
        deg = d0_ref[...] + d1_ref[...] + 1.0
        dinv = lax.rsqrt(deg)
        dinv_ref[...] = dinv
        h = jnp.dot(x_ref[...], w_ref[...], preferred_element_type=jnp.float32)
        g_ref[...] = h * dinv

    return pl.pallas_call(
        body,
        grid=_GRID,
        in_specs=[_row_spec(), _full_spec((D, D)), _row_spec(1), _row_spec(1)],
        out_specs=[_row_spec(), _row_spec(1)],
        out_shape=[
            jax.ShapeDtypeStruct((N, D), jnp.float32),
            jax.ShapeDtypeStruct((N, 1), jnp.float32),
        ],
    )(x, W1, d0, d1)


def _p_spec():
    return pl.BlockSpec((NC, BLK, D), lambda *idx: (0, idx[-1], 0))


def _bn_layer(p, g, dinv, b, gamma, beta, W):
    def body(p_ref, g_ref, dinv_ref, b_ref, ga_ref, be_ref, w_ref, out_ref,
             z_ref, st_ref):
        ph = pl.program_id(0)
        i = pl.program_id(1)

        @pl.when(ph == 0)
        def _():
            z = dinv_ref[...] * (p_ref[0] + p_ref[1] - g_ref[...]) + b_ref[...]
            z_ref[pl.ds(i * BLK, BLK), :] = z

            @pl.when(i == 0)
            def _():
                st_ref[...] = jnp.zeros_like(st_ref)

            st_ref[0:1, :] += jnp.sum(z, axis=0, keepdims=True)
            st_ref[1:2, :] += jnp.sum(z * z, axis=0, keepdims=True)

        @pl.when(ph == 1)
        def _():
            mu = st_ref[0:1, :] * (1.0 / N)
            var = st_ref[1:2, :] * (1.0 / N) - mu * mu
            z = z_ref[pl.ds(i * BLK, BLK), :]
            t = (z - mu) * lax.rsqrt(var + EPS) * ga_ref[...] + be_ref[...]
            t = jnp.maximum(t, 0.0)
            h = jnp.dot(t, w_ref[...], preferred_element_type=jnp.float32)
            out_ref[...] = h * dinv_ref[...]

    return pl.pallas_call(
        body,
        grid=(2,) + _GRID,
        in_specs=[_p_spec(),
                  pl.BlockSpec((BLK, D), lambda ph, i: (i, 0)),
                  pl.BlockSpec((BLK, 1), lambda ph, i: (i, 0)),
                  pl.BlockSpec((1, D), lambda ph, i: (0, 0)),
                  pl.BlockSpec((1, D), lambda ph, i: (0, 0)),
                  pl.BlockSpec((1, D), lambda ph, i: (0, 0)),
                  pl.BlockSpec((D, D), lambda ph, i: (0, 0))],
        out_specs=pl.BlockSpec((BLK, D), lambda ph, i: (i, 0)),
        out_shape=jax.ShapeDtypeStruct((N, D), jnp.float32),
        scratch_shapes=[
            pltpu.VMEM((N, D), jnp.float32),
            pltpu.VMEM((8, D), jnp.float32),
        ],
    )(p, g, dinv, b, gamma, beta, W)


def _final(p, g, dinv, b):
    def body(p_ref, g_ref, dinv_ref, b_ref, o_ref):
        z = dinv_ref[...] * (p_ref[0] + p_ref[1] - g_ref[...]) + b_ref[...]
        o_ref[...] = jax.nn.sigmoid(z)

    return pl.pallas_call(
        body,
        grid=_GRID,
        in_specs=[_p_spec(), _row_spec(), _row_spec(1), _full_spec((1, D))],
        out_specs=_row_spec(),
        out_shape=jax.ShapeDtypeStruct((N, D), jnp.float32),
    )(p, g, dinv, b)


def kernel(x, edge_index, W1, b1, gamma1, beta1, W2, b2, gamma2, beta2, W3, b3):
    src2d = edge_index[0].reshape(IROWS, K)
    dst2d = edge_index[1].reshape(IROWS, K)
    zeros80 = jnp.zeros((HROWS, D), jnp.float32)
    lin80 = jnp.arange(HROWS, dtype=jnp.int32)
    b1r, b2r, b3r = (v.reshape(1, D) for v in (b1, b2, b3))
    ga1, be1 = gamma1.reshape(1, D), beta1.reshape(1, D)
    ga2, be2 = gamma2.reshape(1, D), beta2.reshape(1, D)

    dp = _deg_kernel(edge_index[1], zeros80, lin80)
    d0 = dp[0].reshape(NPAD, 1)[:N]
    d1 = dp[1].reshape(NPAD, 1)[:N]
    g1, dinv = _proj1(x, W1, d0, d1)

    p = _agg_kernel(g1, src2d, dst2d)
    g2 = _bn_layer(p, g1, dinv, b1r, ga1, be1, W2)

    p = _agg_kernel(g2, src2d, dst2d)
    g3 = _bn_layer(p, g2, dinv, b2r, ga2, be2, W3)

    p = _agg_kernel(g3, src2d, dst2d)
    return _final(p, g3, dinv, b3r)
